# Initial kernel scaffold; baseline (speedup 1.0000x reference)
import jax
import jax.numpy as jnp
from jax.experimental import pallas as pl


def kernel(query, knowledge_base, top_k, W1, b1, W2, b2):
    raise NotImplementedError("write your pallas kernel here")


# speedup vs baseline: 4.8337x; 4.8337x over previous
import functools

import jax
import jax.numpy as jnp
from jax import lax
from jax.experimental import pallas as pl
from jax.experimental.pallas import tpu as pltpu
from jax.experimental.pallas import tpu_sc as plsc


def _score_topk_kernel(TN, B, N, D, K,
                       query_ref, w1q_ref, w1k_ref, b1_ref, w2_ref, b2_ref,
                       kb_ref, ts_ref, ti_ref, flat_ref,
                       qproj_scr, scores_scr):
    b = pl.program_id(0)
    j = pl.program_id(1)

    @pl.when((b == 0) & (j == 0))
    def _():
        qproj_scr[...] = (
            jnp.dot(query_ref[...], w1q_ref[...],
                    preferred_element_type=jnp.float32)
            + b1_ref[...])

    kb16 = kb_ref[0].astype(jnp.bfloat16)
    z = jnp.dot(kb16, w1k_ref[...], preferred_element_type=jnp.float32)
    z = z + qproj_scr[pl.ds(b, 1), :]
    h = (0.5 * z * (1.0 + lax.erf(z * 0.7071067811865476))).astype(jnp.bfloat16)
    s = jnp.dot(h, w2_ref[...], preferred_element_type=jnp.float32)
    scores_scr[pl.ds(b, 1), pl.ds(j * TN, TN)] = s.reshape(1, TN) + b2_ref[...]

    @pl.when((b == B - 1) & (j == N // TN - 1))
    def _():
        lane_iota = lax.broadcasted_iota(jnp.int32, (B, N), 1)
        k_iota = lax.broadcasted_iota(jnp.int32, (B, K), 1)

        def body(i, carry):
            ts, ti = carry
            sc = scores_scr[...]
            m = jnp.max(sc, axis=1, keepdims=True)
            idx = jnp.min(jnp.where(sc == m, lane_iota, N),
                          axis=1, keepdims=True)
            ts = jnp.where(k_iota == i, m, ts)
            ti = jnp.where(k_iota == i, idx, ti)
            scores_scr[...] = jnp.where(lane_iota == idx, -jnp.inf, sc)
            return ts, ti

        ts, ti = lax.fori_loop(
            0, K, body,
            (jnp.zeros((B, K), jnp.float32), jnp.zeros((B, K), jnp.int32)))
        ts_ref[...] = ts
        ti_ref[...] = ti
        flat_ref[...] = ti + lax.broadcasted_iota(jnp.int32, (B, K), 0) * N


def _score_topk(query, kb, w1q, w1k, b1, w2, b2, K, TN=1024):
    B, N, D = kb.shape
    grid = (B, N // TN)
    return pl.pallas_call(
        functools.partial(_score_topk_kernel, TN, B, N, D, K),
        grid=grid,
        in_specs=[
            pl.BlockSpec((B, D), lambda b, j: (0, 0)),
            pl.BlockSpec((D, D), lambda b, j: (0, 0)),
            pl.BlockSpec((D, D), lambda b, j: (0, 0)),
            pl.BlockSpec((1, D), lambda b, j: (0, 0)),
            pl.BlockSpec((D, 1), lambda b, j: (0, 0)),
            pl.BlockSpec((1, 1), lambda b, j: (0, 0)),
            pl.BlockSpec((1, TN, D), lambda b, j: (b, j, 0)),
        ],
        out_specs=[
            pl.BlockSpec((B, K), lambda b, j: (0, 0)),
            pl.BlockSpec((B, K), lambda b, j: (0, 0)),
            pl.BlockSpec((B, K), lambda b, j: (0, 0)),
        ],
        out_shape=[
            jax.ShapeDtypeStruct((B, K), jnp.float32),
            jax.ShapeDtypeStruct((B, K), jnp.int32),
            jax.ShapeDtypeStruct((B, K), jnp.int32),
        ],
        scratch_shapes=[
            pltpu.VMEM((B, D), jnp.float32),
            pltpu.VMEM((B, N), jnp.float32),
        ],
        compiler_params=pltpu.CompilerParams(
            dimension_semantics=("arbitrary", "arbitrary")),
    )(query, w1q, w1k, b1, w2, b2, kb)


def _gather_rows(table, flat_idx):
    R, D = flat_idx.shape[0], table.shape[1]
    info = plsc.get_sparse_core_info()
    nw = info.num_cores * info.num_subcores
    bpw = R // nw
    mesh = plsc.VectorSubcoreMesh(core_axis_name="c", subcore_axis_name="s")

    @functools.partial(
        pl.kernel,
        out_type=jax.ShapeDtypeStruct((R, D), jnp.float32),
        mesh=mesh,
        scratch_types=[
            pltpu.VMEM((bpw,), jnp.int32),
            pltpu.VMEM((bpw, D), jnp.float32),
            pltpu.SemaphoreType.DMA,
        ],
    )
    def gk(table_hbm, idx_hbm, out_hbm, idx_v, rows_v, sem):
        wid = lax.axis_index("s") * info.num_cores + lax.axis_index("c")
        base = wid * bpw
        pltpu.sync_copy(idx_hbm.at[pl.ds(base, bpw)], idx_v)
        pltpu.async_copy(table_hbm.at[idx_v], rows_v, sem).wait()
        pltpu.sync_copy(rows_v, out_hbm.at[pl.ds(base, bpw)])

    return gk(table, flat_idx)


def kernel(query, knowledge_base, top_k, W1, b1, W2, b2):
    B, N, D = knowledge_base.shape
    K = min(32, N)
    w1q = W1[:D].astype(jnp.bfloat16)
    w1k = W1[D:].astype(jnp.bfloat16)
    ts, ti, flat = _score_topk(query.astype(jnp.bfloat16), knowledge_base,
                               w1q, w1k, b1.reshape(1, D),
                               W2.astype(jnp.bfloat16), b2.reshape(1, 1), K)
    table = knowledge_base.reshape(B * N, D)
    premises = _gather_rows(table, flat.reshape(B * K)).reshape(B, K, D)
    return premises, ts, ti


# speedup vs baseline: 4.8406x; 1.0014x over previous
import functools

import jax
import jax.numpy as jnp
from jax import lax
from jax.experimental import pallas as pl
from jax.experimental.pallas import tpu as pltpu
from jax.experimental.pallas import tpu_sc as plsc


def _score_topk_kernel(TN, NC, B, N, D, K,
                       query_ref, w1q_ref, w1k_ref, b1_ref, w2_ref, b2_ref,
                       kb_ref, ts_ref, ti_ref, flat_ref,
                       qproj_scr, scol_scr):
    b = pl.program_id(0)
    j = pl.program_id(1)
    CH = TN // NC

    @pl.when((b == 0) & (j == 0))
    def _():
        qproj_scr[...] = (
            jnp.dot(query_ref[...], w1q_ref[...],
                    preferred_element_type=jnp.float32)
            + b1_ref[...])

    kb16 = kb_ref[0].astype(jnp.bfloat16)
    qrow = qproj_scr[pl.ds(b, 1), :]
    for c in range(NC):
        z = jnp.dot(kb16[c * CH:(c + 1) * CH], w1k_ref[...],
                    preferred_element_type=jnp.float32)
        z = z + qrow
        h = (0.5 * z * (1.0 + lax.erf(z * 0.7071067811865476))
             ).astype(jnp.bfloat16)
        s = jnp.dot(h, w2_ref[...], preferred_element_type=jnp.float32)
        rows = pl.ds(j * TN + c * CH, CH)
        ch_lanes = lax.broadcasted_iota(jnp.int32, (CH, B), 1)
        sbc = jnp.broadcast_to(s + b2_ref[...], (CH, B))
        scol_scr[rows, :] = jnp.where(ch_lanes == b, sbc, scol_scr[rows, :])

    @pl.when((b == B - 1) & (j == N // TN - 1))
    def _():
        lane_iota = lax.broadcasted_iota(jnp.int32, (B, N), 1)
        k_iota = lax.broadcasted_iota(jnp.int32, (B, K), 1)

        def body(i, carry):
            sc, ts, ti = carry
            m = jnp.max(sc, axis=1, keepdims=True)
            idx = jnp.min(jnp.where(sc == m, lane_iota, N),
                          axis=1, keepdims=True)
            ts = jnp.where(k_iota == i, m, ts)
            ti = jnp.where(k_iota == i, idx, ti)
            sc = jnp.where(lane_iota == idx, -jnp.inf, sc)
            return sc, ts, ti

        sc0 = scol_scr[...].T
        _, ts, ti = lax.fori_loop(
            0, K, body,
            (sc0, jnp.zeros((B, K), jnp.float32), jnp.zeros((B, K), jnp.int32)))
        ts_ref[...] = ts
        ti_ref[...] = ti
        flat_ref[...] = ti + lax.broadcasted_iota(jnp.int32, (B, K), 0) * N


def _score_topk(query, kb, w1q, w1k, b1, w2, b2, K, TN=1024, NC=2):
    B, N, D = kb.shape
    grid = (B, N // TN)
    return pl.pallas_call(
        functools.partial(_score_topk_kernel, TN, NC, B, N, D, K),
        grid=grid,
        in_specs=[
            pl.BlockSpec((B, D), lambda b, j: (0, 0)),
            pl.BlockSpec((D, D), lambda b, j: (0, 0)),
            pl.BlockSpec((D, D), lambda b, j: (0, 0)),
            pl.BlockSpec((1, D), lambda b, j: (0, 0)),
            pl.BlockSpec((D, 1), lambda b, j: (0, 0)),
            pl.BlockSpec((1, 1), lambda b, j: (0, 0)),
            pl.BlockSpec((1, TN, D), lambda b, j: (b, j, 0)),
        ],
        out_specs=[
            pl.BlockSpec((B, K), lambda b, j: (0, 0)),
            pl.BlockSpec((B, K), lambda b, j: (0, 0)),
            pl.BlockSpec((B, K), lambda b, j: (0, 0)),
        ],
        out_shape=[
            jax.ShapeDtypeStruct((B, K), jnp.float32),
            jax.ShapeDtypeStruct((B, K), jnp.int32),
            jax.ShapeDtypeStruct((B, K), jnp.int32),
        ],
        scratch_shapes=[
            pltpu.VMEM((B, D), jnp.float32),
            pltpu.VMEM((N, B), jnp.float32),
        ],
        compiler_params=pltpu.CompilerParams(
            dimension_semantics=("arbitrary", "arbitrary")),
    )(query, w1q, w1k, b1, w2, b2, kb)


def _gather_rows(table, flat_idx):
    R, D = flat_idx.shape[0], table.shape[1]
    info = plsc.get_sparse_core_info()
    nw = info.num_cores * info.num_subcores
    bpw = R // nw
    mesh = plsc.VectorSubcoreMesh(core_axis_name="c", subcore_axis_name="s")

    @functools.partial(
        pl.kernel,
        out_type=jax.ShapeDtypeStruct((R, D), jnp.float32),
        mesh=mesh,
        scratch_types=[
            pltpu.VMEM((bpw,), jnp.int32),
            pltpu.VMEM((bpw, D), jnp.float32),
            pltpu.SemaphoreType.DMA,
        ],
    )
    def gk(table_hbm, idx_hbm, out_hbm, idx_v, rows_v, sem):
        wid = lax.axis_index("s") * info.num_cores + lax.axis_index("c")
        base = wid * bpw
        pltpu.sync_copy(idx_hbm.at[pl.ds(base, bpw)], idx_v)
        pltpu.async_copy(table_hbm.at[idx_v], rows_v, sem).wait()
        pltpu.sync_copy(rows_v, out_hbm.at[pl.ds(base, bpw)])

    return gk(table, flat_idx)


def kernel(query, knowledge_base, top_k, W1, b1, W2, b2):
    B, N, D = knowledge_base.shape
    K = min(32, N)
    w1q = W1[:D].astype(jnp.bfloat16)
    w1k = W1[D:].astype(jnp.bfloat16)
    ts, ti, flat = _score_topk(query.astype(jnp.bfloat16), knowledge_base,
                               w1q, w1k, b1.reshape(1, D),
                               W2.astype(jnp.bfloat16), b2.reshape(1, 1), K)
    table = knowledge_base.reshape(B * N, D)
    premises = _gather_rows(table, flat.reshape(B * K)).reshape(B, K, D)
    return premises, ts, ti
